# baseline (speedup 1.0000x reference)
import functools

import jax
import jax.numpy as jnp
from jax import lax
from jax.experimental import pallas as pl
from jax.experimental.pallas import tpu as pltpu
from jax.experimental.pallas import tpu_sc as plsc

_L = 16


def kernel(x, table):
    B = x.shape[0]
    V, D = table.shape
    info = plsc.get_sparse_core_info()
    NC, NS = info.num_cores, info.num_subcores
    NW = NC * NS
    b_per_w = B // NW
    n_chunks = b_per_w // _L
    mesh = plsc.VectorSubcoreMesh(core_axis_name="c", subcore_axis_name="s")

    @functools.partial(
        pl.kernel,
        mesh=mesh,
        compiler_params=pltpu.CompilerParams(
            needs_layout_passes=False, use_tc_tiling_on_sc=True
        ),
        out_type=jax.ShapeDtypeStruct((B, D), jnp.float32),
        scratch_types=[
            pltpu.VMEM((b_per_w,), jnp.int32),
            pltpu.VMEM((2, _L * 8, D), jnp.float32),
            pltpu.VMEM((b_per_w, D), jnp.float32),
            pltpu.SemaphoreType.DMA,
            pltpu.SemaphoreType.DMA,
        ],
    )
    def _emb(x_hbm, table_hbm, out_hbm, idx_v, sb, out_v, sem0, sem1):
        wid = lax.axis_index("s") * NC + lax.axis_index("c")
        base = wid * b_per_w
        pltpu.sync_copy(x_hbm.at[pl.ds(base, b_per_w)], idx_v)
        iota = lax.iota(jnp.int32, _L)

        def fire(j, p, sem):
            idx16 = idx_v[pl.ds(j * _L, _L)]
            for l in range(_L):
                row0 = pl.multiple_of(idx16[l] & ~jnp.int32(7), 8)
                pltpu.async_copy(
                    table_hbm.at[pl.ds(row0, 8)],
                    sb.at[p, pl.ds(l * 8, 8)],
                    sem,
                )

        def drain(p, sem):
            for l in range(_L):
                pltpu.make_async_copy(
                    table_hbm.at[pl.ds(0, 8)], sb.at[p, pl.ds(l * 8, 8)], sem
                ).wait()

        def extract(j, p):
            idx16 = idx_v[pl.ds(j * _L, _L)]
            r_vec = iota * 8 + (idx16 & 7)
            i_vec = j * _L + iota
            buf = sb.at[p]
            for c in range(D):
                col = jnp.full((_L,), c, jnp.int32)
                vals = plsc.load_gather(buf, [r_vec, col])
                plsc.store_scatter(out_v, [i_vec, col], vals)

        fire(0, 0, sem0)

        def pair(jp, carry):
            j0 = 2 * jp
            fire(j0 + 1, 1, sem1)
            drain(0, sem0)
            extract(j0, 0)

            @pl.when(jp + 1 < n_chunks // 2)
            def _():
                fire(j0 + 2, 0, sem0)

            drain(1, sem1)
            extract(j0 + 1, 1)
            return carry

        lax.fori_loop(0, n_chunks // 2, pair, 0)
        pltpu.sync_copy(out_v, out_hbm.at[pl.ds(base, b_per_w)])

    return _emb(x, table)


# speedup vs baseline: 1.1020x; 1.1020x over previous
import functools

import jax
import jax.numpy as jnp
from jax import lax
from jax.experimental import pallas as pl
from jax.experimental.pallas import tpu as pltpu
from jax.experimental.pallas import tpu_sc as plsc

_L = 16


def kernel(x, table):
    B = x.shape[0]
    V, D = table.shape
    info = plsc.get_sparse_core_info()
    NC, NS = info.num_cores, info.num_subcores
    NW = NC * NS
    b_per_w = B // NW
    mesh = plsc.VectorSubcoreMesh(core_axis_name="c", subcore_axis_name="s")

    @functools.partial(
        pl.kernel,
        mesh=mesh,
        compiler_params=pltpu.CompilerParams(
            needs_layout_passes=False, use_tc_tiling_on_sc=True
        ),
        out_type=jax.ShapeDtypeStruct((B, D), jnp.float32),
        scratch_types=[
            pltpu.VMEM((b_per_w,), jnp.int32),
            pltpu.VMEM((b_per_w, D), jnp.float32),
            pltpu.SemaphoreType.DMA,
        ],
    )
    def _emb(x_hbm, table_hbm, out_hbm, idx_v, out_v, sem):
        wid = lax.axis_index("s") * NC + lax.axis_index("c")
        base = wid * b_per_w
        pltpu.sync_copy(x_hbm.at[pl.ds(base, b_per_w)], idx_v)

        def body(g, carry):
            idx16 = idx_v[pl.ds(g * _L, _L)]
            for l in range(_L):
                idx = idx16[l]
                pltpu.async_copy(
                    table_hbm.at[pl.ds(idx, 1)],
                    out_v.at[pl.ds(g * _L + l, 1)],
                    sem,
                )
            return carry

        lax.fori_loop(0, b_per_w // _L, body, 0)
        pltpu.make_async_copy(
            table_hbm.at[pl.ds(0, b_per_w)], out_v, sem
        ).wait()
        pltpu.sync_copy(out_v, out_hbm.at[pl.ds(base, b_per_w)])

    return _emb(x, table)
